# baseline (speedup 1.0000x reference)
import functools

import jax
import jax.numpy as jnp
from jax import lax
from jax.experimental import pallas as pl
from jax.experimental.pallas import tpu as pltpu
from jax.experimental.pallas import tpu_sc as plsc

_VOCAB = 1000000
_EMBED = 64
_BATCH = 16384
_NW = 32
_BS = 128
_NBF = 7812
_TAIL0 = _NBF * _BS
_NBW = 245
_NSLOT = 5
_QUADS = (_NBW + _NSLOT - 1) // _NSLOT
_NIDX = 2 * _BATCH
_HCAP = _NIDX + 16
_SCAP = 64
_FLUSH_AT = _SCAP - 16


def _make_phase_a():
    mesh = plsc.VectorSubcoreMesh(core_axis_name="c", subcore_axis_name="s")

    @functools.partial(
        pl.kernel,
        mesh=mesh,
        compiler_params=pltpu.CompilerParams(
            needs_layout_passes=False, use_tc_tiling_on_sc=True),
        out_type=jax.ShapeDtypeStruct((_NIDX, 128), jnp.float32),
        scratch_types=[
            pltpu.VMEM((2048,), jnp.int32),
            pltpu.VMEM((_HCAP,), jnp.int32),
            pltpu.VMEM((_HCAP,), jnp.int32),
            pltpu.VMEM((_EMBED, _BS), jnp.float32),
            pltpu.VMEM((_EMBED, _BS), jnp.float32),
            pltpu.VMEM((_EMBED, _BS), jnp.float32),
            pltpu.VMEM((_EMBED, _BS), jnp.float32),
            pltpu.VMEM((_EMBED, _BS), jnp.float32),
            pltpu.VMEM((_EMBED, 128), jnp.float32),
            pltpu.VMEM((_SCAP, 128), jnp.float32),
            pltpu.VMEM((_SCAP,), jnp.int32),
            pltpu.SMEM((17,), jnp.int32),
            pltpu.SMEM((1024,), jnp.int32),
            pltpu.SemaphoreType.DMA,
            pltpu.SemaphoreType.DMA,
            pltpu.SemaphoreType.DMA,
            pltpu.SemaphoreType.DMA,
            pltpu.SemaphoreType.DMA,
            pltpu.SemaphoreType.DMA,
        ],
    )
    def phase_a(widx_hbm, cidx_hbm, tablet_hbm, tail_hbm, rows_hbm,
                idxc, hits, bkt, tb0, tb1, tb2, tb3, tb4, tail_v,
                big, posr, sm, cnsm,
                semd0, semd1, semd2, semd3, semd4, semf):
        wid = lax.axis_index("s") * 2 + lax.axis_index("c")
        jlo = wid * _NBW
        lo = jlo * _BS
        hi = jnp.minimum(lo + _NBW * _BS, _VOCAB)
        iota16 = lax.iota(jnp.int32, 16)
        evs = [iota16 + 16 * k for k in range(4)]

        pltpu.sync_copy(tail_hbm, tail_v)

        bufs = [tb0, tb1, tb2, tb3, tb4]
        sems = [semd0, semd1, semd2, semd3, semd4]

        def fire(slot, j):
            jc = jnp.minimum(j, _NBF - 1)
            off = pl.multiple_of(jc * _BS, _BS)
            return pltpu.async_copy(tablet_hbm.at[:, pl.ds(off, _BS)],
                                    bufs[slot], sems[slot])

        for s4 in range(_NSLOT):
            fire(s4, jlo + s4)

        def compact(nv, maskfn, valfn, dst, cursor):
            nsub = (nv + 1023) >> 10

            def sub(si, cur):
                vbase = si * 1024
                nvh = jnp.minimum(1024, nv - vbase)

                def p1(vv, _):
                    cnt = plsc.all_reduce_population_count(
                        maskfn(vbase + vv))
                    cnsm[vv] = jnp.max(cnt)
                    return 0

                lax.fori_loop(0, nvh, p1, 0)

                def p2(vv, cur):
                    v = vbase + vv
                    plsc.store_compressed(dst.at[pl.ds(cur, 16)], valfn(v),
                                          mask=maskfn(v))
                    return cur + cnsm[vv]

                return lax.fori_loop(0, nvh, p2, cur)

            return lax.fori_loop(0, nsub, sub, cursor)

        nh = jnp.int32(0)
        for c in range(_NIDX // 2048):
            src_hbm = widx_hbm if c < _BATCH // 2048 else cidx_hbm
            pltpu.sync_copy(
                src_hbm.at[pl.ds((c * 2048) % _BATCH, 2048)], idxc)

            def maskfn(v):
                r = idxc[pl.ds(v * 16, 16)]
                return (r >= lo) & (r < hi)

            def valfn(v, c=c):
                r = idxc[pl.ds(v * 16, 16)]
                pos = (c * 2048 + v * 16) + iota16
                return (pos << 15) | (r - lo)

            nh = compact(128, maskfn, valfn, hits, nh)

        nv = (nh + 15) >> 4
        cur = jnp.int32(0)
        for b in range(16):
            sm[b] = cur

            def maskfn(v, b=b):
                h = hits[pl.ds(v * 16, 16)]
                valid = (v * 16 + iota16) < nh
                return valid & (((h & 0x7FFF) >> 11) == b)

            def valfn(v):
                return hits[pl.ds(v * 16, 16)]

            cur = compact(nv, maskfn, valfn, bkt, cur)
        sm[16] = cur

        match = hits

        def flush():
            pltpu.async_copy(
                big, rows_hbm.at[plsc.Indices(posr, ignored_value=-1)],
                semf).wait()
            neg = jnp.full((16,), -1, jnp.int32)
            for q in range(_SCAP // 16):
                posr[pl.ds(q * 16, 16)] = neg

        def process_block(j, jok, src, cursor):
            jrel = j - jlo
            b = jrel >> 4
            s = sm[b]
            t = sm[b + 1]
            v0 = s >> 4
            nv = ((t + 15) >> 4) - v0

            def mmask(vv):
                v = v0 + vv
                h = bkt[pl.ds(v * 16, 16)]
                k = v * 16 + iota16
                return jok & (k >= s) & (k < t) & (
                    ((h & 0x7FFF) >> 7) == jrel)

            def mval(vv):
                return bkt[pl.ds((v0 + vv) * 16, 16)]

            nm = compact(nv, mmask, mval, match, jnp.int32(0))

            def ebody(g, cur):
                cur = lax.cond(cur > _FLUSH_AT,
                               lambda: (flush(), jnp.int32(0))[1],
                               lambda: cur)
                h = match[pl.ds(g * 16, 16)]
                valid = (g * 16 + iota16) < nm
                posr[pl.ds(cur, 16)] = jnp.where(valid, h >> 15, -1)
                nmg = jnp.minimum(16, nm - g * 16)

                def hbody(i, _, g=g):
                    hsp = plsc.load_gather(
                        match, [jnp.full((16,), 0, jnp.int32) + (g * 16 + i)])
                    lane = hsp & 127
                    for k in range(4):
                        vals = plsc.load_gather(src, [evs[k], lane])
                        big[cur + i, pl.ds(k * 16, 16)] = vals
                    return 0

                lax.fori_loop(0, nmg, hbody, 0)
                return cur + 16

            return lax.fori_loop(0, (nm + 15) >> 4, ebody, cursor)

        neg = jnp.full((16,), -1, jnp.int32)
        for q in range(_SCAP // 16):
            posr[pl.ds(q * 16, 16)] = neg

        def quad(ii, cursor):
            j0 = jlo + _NSLOT * ii
            for s4 in range(_NSLOT):
                pltpu.make_async_copy(
                    tablet_hbm.at[:, pl.ds(pl.multiple_of(0, _BS), _BS)],
                    bufs[s4], sems[s4]).wait()
                cursor = process_block(j0 + s4, (j0 + s4) < _NBF,
                                       bufs[s4], cursor)
                fire(s4, j0 + s4 + _NSLOT)
            return cursor

        cursor = lax.fori_loop(0, _QUADS, quad, jnp.int32(0))

        cursor = lax.cond(wid == _NW - 1,
                          lambda c: process_block(jnp.int32(_NBF), True,
                                                  tail_v, c),
                          lambda c: c, cursor)

        flush()

        dummy = tablet_hbm.at[:, pl.ds(pl.multiple_of(0, _BS), _BS)]
        for s4 in range(_NSLOT):
            pltpu.make_async_copy(dummy, bufs[s4], sems[s4]).wait()

    return phase_a


def _phase_b_body(wref, cref, wscal, bscal, oref):
    s = jnp.sum((wref[...] * cref[...])[:, :_EMBED], axis=1, keepdims=True)
    z = s * wscal[0, 0] + bscal[0, 0]
    oref[...] = 1.0 / (1.0 + jnp.exp(-z))


def _make_phase_b():
    blk = 8192
    grid = _BATCH // blk
    return pl.pallas_call(
        _phase_b_body,
        grid=(grid,),
        in_specs=[
            pl.BlockSpec((blk, 128), lambda i: (i, 0)),
            pl.BlockSpec((blk, 128), lambda i: (i + grid, 0)),
            pl.BlockSpec((1, 1), lambda i: (0, 0), memory_space=pltpu.SMEM),
            pl.BlockSpec((1, 1), lambda i: (0, 0), memory_space=pltpu.SMEM),
        ],
        out_specs=pl.BlockSpec((blk, 1), lambda i: (i, 0)),
        out_shape=jax.ShapeDtypeStruct((_BATCH, 1), jnp.float32),
        compiler_params=pltpu.CompilerParams(
            dimension_semantics=("arbitrary",)),
    )


_phase_a = _make_phase_a()
_phase_b = _make_phase_b()


@jax.jit
def kernel(word, context, table, dense_w, dense_b):
    widx = word.reshape(_BATCH).astype(jnp.int32)
    cidx = context.reshape(_BATCH).astype(jnp.int32)
    tablet = table.T
    tail = jnp.pad(table[_TAIL0:].T.astype(jnp.float32), ((0, 0), (0, 64)))
    rows = _phase_a(widx, cidx, tablet, tail)
    out = _phase_b(rows, rows,
                   dense_w.reshape(1, 1).astype(jnp.float32),
                   dense_b.reshape(1, 1).astype(jnp.float32))
    return out


# speedup vs baseline: 1.0142x; 1.0142x over previous
import functools

import jax
import jax.numpy as jnp
from jax import lax
from jax.experimental import pallas as pl
from jax.experimental.pallas import tpu as pltpu
from jax.experimental.pallas import tpu_sc as plsc

_VOCAB = 1000000
_EMBED = 64
_BATCH = 16384
_NW = 32
_BS = 128
_NBF = 7812
_TAIL0 = _NBF * _BS
_NBW = 245
_NSLOT = 5
_QUADS = (_NBW + _NSLOT - 1) // _NSLOT
_NIDX = 2 * _BATCH
_HCAP = _NIDX + 16
_SCAP = 64
_FLUSH_AT = _SCAP - 16


def _make_phase_a():
    mesh = plsc.VectorSubcoreMesh(core_axis_name="c", subcore_axis_name="s")

    @functools.partial(
        pl.kernel,
        mesh=mesh,
        compiler_params=pltpu.CompilerParams(
            needs_layout_passes=False, use_tc_tiling_on_sc=True),
        out_type=jax.ShapeDtypeStruct((_NIDX, 128), jnp.float32),
        scratch_types=[
            pltpu.VMEM((2048,), jnp.int32),
            pltpu.VMEM((_HCAP,), jnp.int32),
            pltpu.VMEM((_HCAP,), jnp.int32),
            pltpu.VMEM((_EMBED, _BS), jnp.float32),
            pltpu.VMEM((_EMBED, _BS), jnp.float32),
            pltpu.VMEM((_EMBED, _BS), jnp.float32),
            pltpu.VMEM((_EMBED, _BS), jnp.float32),
            pltpu.VMEM((_EMBED, _BS), jnp.float32),
            pltpu.VMEM((_EMBED, 128), jnp.float32),
            pltpu.VMEM((_SCAP, 128), jnp.float32),
            pltpu.VMEM((_SCAP,), jnp.int32),
            pltpu.SMEM((17,), jnp.int32),
            pltpu.SMEM((1024,), jnp.int32),
            pltpu.SemaphoreType.DMA,
            pltpu.SemaphoreType.DMA,
            pltpu.SemaphoreType.DMA,
            pltpu.SemaphoreType.DMA,
            pltpu.SemaphoreType.DMA,
            pltpu.SemaphoreType.DMA,
        ],
    )
    def phase_a(widx_hbm, cidx_hbm, tablet_hbm, tail_hbm, rows_hbm,
                idxc, hits, bkt, tb0, tb1, tb2, tb3, tb4, tail_v,
                big, posr, sm, cnsm,
                semd0, semd1, semd2, semd3, semd4, semf):
        wid = lax.axis_index("s") * 2 + lax.axis_index("c")
        jlo = wid * _NBW
        lo = jlo * _BS
        hi = jnp.minimum(lo + _NBW * _BS, _VOCAB)
        iota16 = lax.iota(jnp.int32, 16)
        evs = [iota16 + 16 * k for k in range(4)]

        pltpu.sync_copy(tail_hbm, tail_v)

        bufs = [tb0, tb1, tb2, tb3, tb4]
        sems = [semd0, semd1, semd2, semd3, semd4]

        def fire(slot, j):
            jc = jnp.minimum(j, _NBF - 1)
            off = pl.multiple_of(jc * _BS, _BS)
            return pltpu.async_copy(tablet_hbm.at[:, pl.ds(off, _BS)],
                                    bufs[slot], sems[slot])

        for s4 in range(_NSLOT):
            fire(s4, jlo + s4)

        def compact(nv, maskfn, valfn, dst, cursor):
            nsub = (nv + 1023) >> 10

            def sub(si, cur):
                vbase = si * 1024
                nvh = jnp.minimum(1024, nv - vbase)

                def p1(vv, _):
                    cnt = plsc.all_reduce_population_count(
                        maskfn(vbase + vv))
                    cnsm[vv] = jnp.max(cnt)
                    return 0

                lax.fori_loop(0, nvh, p1, 0)

                def p2(vv, cur):
                    v = vbase + vv
                    plsc.store_compressed(dst.at[pl.ds(cur, 16)], valfn(v),
                                          mask=maskfn(v))
                    return cur + cnsm[vv]

                return lax.fori_loop(0, nvh, p2, cur)

            return lax.fori_loop(0, nsub, sub, cursor)

        nh = jnp.int32(0)
        for c in range(_NIDX // 2048):
            src_hbm = widx_hbm if c < _BATCH // 2048 else cidx_hbm
            pltpu.sync_copy(
                src_hbm.at[pl.ds((c * 2048) % _BATCH, 2048)], idxc)

            def maskfn(v):
                r = idxc[pl.ds(v * 16, 16)]
                return (r >= lo) & (r < hi)

            def valfn(v, c=c):
                r = idxc[pl.ds(v * 16, 16)]
                pos = (c * 2048 + v * 16) + iota16
                return (pos << 15) | (r - lo)

            nh = compact(128, maskfn, valfn, hits, nh)

        nv = (nh + 15) >> 4
        cur = jnp.int32(0)
        csm_b = [jnp.int32(0)] * 5
        for cq in range(4):
            csm_b[cq] = cur

            def maskfn(v, cq=cq):
                h = hits[pl.ds(v * 16, 16)]
                valid = (v * 16 + iota16) < nh
                return valid & (((h & 0x7FFF) >> 13) == cq)

            def valfn(v):
                return hits[pl.ds(v * 16, 16)]

            cur = compact(nv, maskfn, valfn, bkt, cur)
        csm_b[4] = cur

        cur = jnp.int32(0)
        for cq in range(4):
            s_c = csm_b[cq]
            t_c = csm_b[cq + 1]
            v0_c = s_c >> 4
            nv_c = ((t_c + 15) >> 4) - v0_c
            for fq in range(4):
                sm[cq * 4 + fq] = cur

                def maskfn(vv, cq=cq, fq=fq, s_c=s_c, t_c=t_c, v0_c=v0_c):
                    v = v0_c + vv
                    h = bkt[pl.ds(v * 16, 16)]
                    k = v * 16 + iota16
                    return ((k >= s_c) & (k < t_c)
                            & ((((h & 0x7FFF) >> 11) & 3) == fq))

                def valfn(vv, v0_c=v0_c):
                    return bkt[pl.ds((v0_c + vv) * 16, 16)]

                cur = compact(nv_c, maskfn, valfn, hits, cur)
        sm[16] = cur

        bktf = hits
        match = bkt

        def flush():
            pltpu.async_copy(
                big, rows_hbm.at[plsc.Indices(posr, ignored_value=-1)],
                semf).wait()
            neg = jnp.full((16,), -1, jnp.int32)
            for q in range(_SCAP // 16):
                posr[pl.ds(q * 16, 16)] = neg

        def process_block(j, jok, src, cursor):
            jrel = j - jlo
            b = jrel >> 4
            s = sm[b]
            t = sm[b + 1]
            v0 = s >> 4
            nv = ((t + 15) >> 4) - v0

            def mmask(vv):
                v = v0 + vv
                h = bktf[pl.ds(v * 16, 16)]
                k = v * 16 + iota16
                return jok & (k >= s) & (k < t) & (
                    ((h & 0x7FFF) >> 7) == jrel)

            def mval(vv):
                return bktf[pl.ds((v0 + vv) * 16, 16)]

            nm = compact(nv, mmask, mval, match, jnp.int32(0))

            def ebody(g, cur):
                cur = lax.cond(cur > _FLUSH_AT,
                               lambda: (flush(), jnp.int32(0))[1],
                               lambda: cur)
                h = match[pl.ds(g * 16, 16)]
                valid = (g * 16 + iota16) < nm
                posr[pl.ds(cur, 16)] = jnp.where(valid, h >> 15, -1)
                nmg = jnp.minimum(16, nm - g * 16)

                def hbody(i, _, g=g):
                    hsp = plsc.load_gather(
                        match, [jnp.full((16,), 0, jnp.int32) + (g * 16 + i)])
                    lane = hsp & 127
                    for k in range(4):
                        vals = plsc.load_gather(src, [evs[k], lane])
                        big[cur + i, pl.ds(k * 16, 16)] = vals
                    return 0

                lax.fori_loop(0, nmg, hbody, 0)
                return cur + 16

            return lax.fori_loop(0, (nm + 15) >> 4, ebody, cursor)

        neg = jnp.full((16,), -1, jnp.int32)
        for q in range(_SCAP // 16):
            posr[pl.ds(q * 16, 16)] = neg

        def quad(ii, cursor):
            j0 = jlo + _NSLOT * ii
            for s4 in range(_NSLOT):
                pltpu.make_async_copy(
                    tablet_hbm.at[:, pl.ds(pl.multiple_of(0, _BS), _BS)],
                    bufs[s4], sems[s4]).wait()
                cursor = process_block(j0 + s4, (j0 + s4) < _NBF,
                                       bufs[s4], cursor)
                fire(s4, j0 + s4 + _NSLOT)
            return cursor

        cursor = lax.fori_loop(0, _QUADS, quad, jnp.int32(0))

        cursor = lax.cond(wid == _NW - 1,
                          lambda c: process_block(jnp.int32(_NBF), True,
                                                  tail_v, c),
                          lambda c: c, cursor)

        flush()

        dummy = tablet_hbm.at[:, pl.ds(pl.multiple_of(0, _BS), _BS)]
        for s4 in range(_NSLOT):
            pltpu.make_async_copy(dummy, bufs[s4], sems[s4]).wait()

    return phase_a


def _phase_b_body(wref, cref, wscal, bscal, oref):
    s = jnp.sum((wref[...] * cref[...])[:, :_EMBED], axis=1, keepdims=True)
    z = s * wscal[0, 0] + bscal[0, 0]
    oref[...] = 1.0 / (1.0 + jnp.exp(-z))


def _make_phase_b():
    blk = 8192
    grid = _BATCH // blk
    return pl.pallas_call(
        _phase_b_body,
        grid=(grid,),
        in_specs=[
            pl.BlockSpec((blk, 128), lambda i: (i, 0)),
            pl.BlockSpec((blk, 128), lambda i: (i + grid, 0)),
            pl.BlockSpec((1, 1), lambda i: (0, 0), memory_space=pltpu.SMEM),
            pl.BlockSpec((1, 1), lambda i: (0, 0), memory_space=pltpu.SMEM),
        ],
        out_specs=pl.BlockSpec((blk, 1), lambda i: (i, 0)),
        out_shape=jax.ShapeDtypeStruct((_BATCH, 1), jnp.float32),
        compiler_params=pltpu.CompilerParams(
            dimension_semantics=("arbitrary",)),
    )


_phase_a = _make_phase_a()
_phase_b = _make_phase_b()


@jax.jit
def kernel(word, context, table, dense_w, dense_b):
    widx = word.reshape(_BATCH).astype(jnp.int32)
    cidx = context.reshape(_BATCH).astype(jnp.int32)
    tablet = table.T
    tail = jnp.pad(table[_TAIL0:].T.astype(jnp.float32), ((0, 0), (0, 64)))
    rows = _phase_a(widx, cidx, tablet, tail)
    out = _phase_b(rows, rows,
                   dense_w.reshape(1, 1).astype(jnp.float32),
                   dense_b.reshape(1, 1).astype(jnp.float32))
    return out
